# baseline (speedup 1.0000x reference)
import functools

import jax
import jax.numpy as jnp
from jax import lax
from jax.experimental import pallas as pl
from jax.experimental.pallas import tpu as pltpu
from jax.experimental.pallas import tpu_sc as plsc

N = 10000
E = 320000
D = 128
C = 64

NC = 2
NS = 16
NW = NC * NS
PT = 640
NPAD = NS * PT
EPW = E // NW
CH = 125
NCHUNK = EPW // CH
EROWS = E // CH

_mesh = plsc.VectorSubcoreMesh(core_axis_name="c", subcore_axis_name="s")


def _deg_body(col_hbm, out_hbm, cidx_v, hist_v):
    c = lax.axis_index("c")
    s = lax.axis_index("s")
    wid = c * NS + s
    pltpu.sync_copy(col_hbm.at[pl.ds(wid * EPW, EPW)], cidx_v)
    zeros16 = jnp.zeros((16,), jnp.float32)
    ones16 = jnp.ones((16,), jnp.float32)

    def zbody(k, carry):
        hist_v[pl.ds(pl.multiple_of(k * 16, 16), 16)] = zeros16
        return carry

    lax.fori_loop(0, NPAD // 16, zbody, 0)

    def body(k, carry):
        idx = cidx_v[pl.ds(pl.multiple_of(k * 16, 16), 16)]
        plsc.addupdate_scatter(hist_v, [idx], ones16)
        return carry

    lax.fori_loop(0, EPW // 16, body, 0)
    pltpu.sync_copy(hist_v, out_hbm.at[wid])


_deg_call = functools.partial(
    pl.kernel,
    out_type=jax.ShapeDtypeStruct((NW, NPAD), jnp.float32),
    mesh=_mesh,
    compiler_params=pltpu.CompilerParams(needs_layout_passes=False),
    scratch_types=[
        pltpu.VMEM((EPW,), jnp.int32),
        pltpu.VMEM((NPAD,), jnp.float32),
    ],
)(_deg_body)


def _prop_body(g_hbm, row2_hbm, col2_hbm, zeros_hbm, out_hbm,
               ridx_v, cidx_v, rows_v, acc_sh, gsem):
    c = lax.axis_index("c")
    s = lax.axis_index("s")
    wid = c * NS + s
    pltpu.sync_copy(zeros_hbm, acc_sh.at[pl.ds(s * PT, PT)])
    pltpu.sync_copy(row2_hbm.at[pl.ds(wid * NCHUNK, NCHUNK), :], ridx_v)
    pltpu.sync_copy(col2_hbm.at[pl.ds(wid * NCHUNK, NCHUNK), :], cidx_v)
    plsc.subcore_barrier()

    def body(j, carry):
        pltpu.async_copy(g_hbm.at[ridx_v.at[j]], rows_v, gsem).wait()
        pltpu.sync_copy(rows_v, acc_sh.at[cidx_v.at[j]], add=True)
        return carry

    lax.fori_loop(0, NCHUNK, body, 0)
    plsc.subcore_barrier()
    pltpu.sync_copy(acc_sh.at[pl.ds(s * PT, PT)],
                    out_hbm.at[c, pl.ds(s * PT, PT), :])


_prop_call = functools.partial(
    pl.kernel,
    out_type=jax.ShapeDtypeStruct((NC, NPAD, D), jnp.float32),
    mesh=_mesh,
    scratch_types=[
        pltpu.VMEM((NCHUNK, CH), jnp.int32),
        pltpu.VMEM((NCHUNK, CH), jnp.int32),
        pltpu.VMEM((CH, D), jnp.float32),
        pltpu.VMEM_SHARED((NPAD, D), jnp.float32),
        pltpu.SemaphoreType.DMA,
    ],
)(_prop_body)


RB = 1024
GRID = (NPAD + RB - 1) // RB


def _scale0_body(deg_ref, x_ref, dinv_ref, g0_ref):
    deg = jnp.sum(deg_ref[...], axis=0) + 1.0
    dinv = lax.rsqrt(deg)
    dinv_ref[...] = dinv
    g0_ref[...] = dinv * x_ref[...]


_scale0 = pl.pallas_call(
    _scale0_body,
    grid=(GRID,),
    in_specs=[
        pl.BlockSpec((NW, RB, 1), lambda i: (0, i, 0)),
        pl.BlockSpec((RB, D), lambda i: (i, 0)),
    ],
    out_specs=[
        pl.BlockSpec((RB, 1), lambda i: (i, 0)),
        pl.BlockSpec((RB, D), lambda i: (i, 0)),
    ],
    out_shape=[
        jax.ShapeDtypeStruct((N, 1), jnp.float32),
        jax.ShapeDtypeStruct((N, D), jnp.float32),
    ],
)


def _scale1_body(dinv_ref, pa_ref, pb_ref, g0_ref, g2_ref):
    dinv = dinv_ref[...]
    h = pa_ref[...] + pb_ref[...] + g0_ref[...]
    g2_ref[...] = h * (dinv * dinv)


_scale1 = pl.pallas_call(
    _scale1_body,
    grid=(GRID,),
    in_specs=[
        pl.BlockSpec((RB, 1), lambda i: (i, 0)),
        pl.BlockSpec((RB, D), lambda i: (i, 0)),
        pl.BlockSpec((RB, D), lambda i: (i, 0)),
        pl.BlockSpec((RB, D), lambda i: (i, 0)),
    ],
    out_specs=pl.BlockSpec((RB, D), lambda i: (i, 0)),
    out_shape=jax.ShapeDtypeStruct((N, D), jnp.float32),
)


def _final_body(dinv_ref, pa_ref, pb_ref, g2_ref, w_ref, b_ref, out_ref):
    h = dinv_ref[...] * (pa_ref[...] + pb_ref[...] + g2_ref[...])
    out_ref[...] = (
        jnp.dot(h, w_ref[...].T, preferred_element_type=jnp.float32)
        + b_ref[...]
    )


_final = pl.pallas_call(
    _final_body,
    grid=(GRID,),
    in_specs=[
        pl.BlockSpec((RB, 1), lambda i: (i, 0)),
        pl.BlockSpec((RB, D), lambda i: (i, 0)),
        pl.BlockSpec((RB, D), lambda i: (i, 0)),
        pl.BlockSpec((RB, D), lambda i: (i, 0)),
        pl.BlockSpec((C, D), lambda i: (0, 0)),
        pl.BlockSpec((1, C), lambda i: (0, 0)),
    ],
    out_specs=pl.BlockSpec((RB, C), lambda i: (i, 0)),
    out_shape=jax.ShapeDtypeStruct((N, C), jnp.float32),
)


def kernel(x, edge_index, W, b):
    ei = edge_index.astype(jnp.int32)
    row2 = ei[0].reshape(EROWS, CH)
    col2 = ei[1].reshape(EROWS, CH)
    zeros_2d = jnp.zeros((PT, D), jnp.float32)

    deg_parts = _deg_call(ei[1]).reshape(NW, NPAD, 1)
    dinv, g0 = _scale0(deg_parts, x)

    p = _prop_call(g0, row2, col2, zeros_2d)
    g2 = _scale1(dinv, p[0], p[1], g0)
    q = _prop_call(g2, row2, col2, zeros_2d)
    return _final(dinv, q[0], q[1], g2, W, b.reshape(1, C))


# speedup vs baseline: 1.2895x; 1.2895x over previous
import functools

import jax
import jax.numpy as jnp
from jax import lax
from jax.experimental import pallas as pl
from jax.experimental.pallas import tpu as pltpu
from jax.experimental.pallas import tpu_sc as plsc

N = 10000
E = 320000
D = 128
C = 64

NC = 2
NS = 16
NW = NC * NS
PT = 640
NPAD = NS * PT
EPW = E // NW
CH = 125
NCHUNK = EPW // CH
EROWS = E // CH

_mesh = plsc.VectorSubcoreMesh(core_axis_name="c", subcore_axis_name="s")


def _deg_body(col_hbm, out_hbm, cidx_v, hist_v):
    c = lax.axis_index("c")
    s = lax.axis_index("s")
    wid = c * NS + s
    pltpu.sync_copy(col_hbm.at[pl.ds(wid * EPW, EPW)], cidx_v)
    zeros16 = jnp.zeros((16,), jnp.float32)
    ones16 = jnp.ones((16,), jnp.float32)

    def zbody(k, carry):
        for u in range(8):
            hist_v[pl.ds(pl.multiple_of((k * 8 + u) * 16, 16), 16)] = zeros16
        return carry

    lax.fori_loop(0, NPAD // 128, zbody, 0)

    def body(k, carry):
        for u in range(5):
            idx = cidx_v[pl.ds(pl.multiple_of((k * 5 + u) * 16, 16), 16)]
            plsc.addupdate_scatter(hist_v, [idx], ones16)
        return carry

    lax.fori_loop(0, EPW // 80, body, 0)
    pltpu.sync_copy(hist_v, out_hbm.at[wid])


_deg_call = functools.partial(
    pl.kernel,
    out_type=jax.ShapeDtypeStruct((NW, NPAD), jnp.float32),
    mesh=_mesh,
    compiler_params=pltpu.CompilerParams(needs_layout_passes=False),
    scratch_types=[
        pltpu.VMEM((EPW,), jnp.int32),
        pltpu.VMEM((NPAD,), jnp.float32),
    ],
)(_deg_body)


SEG = NCHUNK // 2


def _prop_body(g_hbm, row2_hbm, col2_hbm, zeros_hbm, out_hbm,
               ridx_v, cidx_v, rows0, rows1, acc_sh, gsem0, gsem1):
    c = lax.axis_index("c")
    s = lax.axis_index("s")
    wid = c * NS + s
    rows = [rows0, rows1]
    gsems = [gsem0, gsem1]

    def issue_gather(blk, slot):
        pltpu.async_copy(g_hbm.at[ridx_v.at[blk]], rows[slot], gsems[slot])

    def wait_gather(blk, slot):
        pltpu.make_async_copy(
            g_hbm.at[ridx_v.at[blk]], rows[slot], gsems[slot]).wait()

    def sync_scatter(blk, slot):
        pltpu.sync_copy(rows[slot], acc_sh.at[cidx_v.at[blk]], add=True)

    pltpu.sync_copy(zeros_hbm, acc_sh.at[pl.ds(s * PT, PT)])
    plsc.subcore_barrier()

    for seg in range(2):
        base = wid * NCHUNK + seg * SEG
        pltpu.sync_copy(row2_hbm.at[pl.ds(base, SEG), :], ridx_v)
        pltpu.sync_copy(col2_hbm.at[pl.ds(base, SEG), :], cidx_v)
        issue_gather(0, 0)
        issue_gather(1, 1)

        def body(i, carry):
            for b in range(2):
                t = 2 * i + b
                wait_gather(t, b)
                sync_scatter(t, b)
                issue_gather(lax.rem(t + 2, SEG), b)
            return carry

        lax.fori_loop(0, SEG // 2, body, 0)
        wait_gather(0, 0)
        wait_gather(1, 1)

    plsc.subcore_barrier()
    pltpu.sync_copy(acc_sh.at[pl.ds(s * PT, PT)],
                    out_hbm.at[c, pl.ds(s * PT, PT), :])


_prop_call = functools.partial(
    pl.kernel,
    out_type=jax.ShapeDtypeStruct((NC, NPAD, D), jnp.float32),
    mesh=_mesh,
    scratch_types=[
        pltpu.VMEM((SEG, CH), jnp.int32),
        pltpu.VMEM((SEG, CH), jnp.int32),
        pltpu.VMEM((CH, D), jnp.float32),
        pltpu.VMEM((CH, D), jnp.float32),
        pltpu.VMEM_SHARED((NPAD, D), jnp.float32),
        pltpu.SemaphoreType.DMA,
        pltpu.SemaphoreType.DMA,
    ],
)(_prop_body)


RB = 1024
GRID = (NPAD + RB - 1) // RB


def _scale0_body(deg_ref, x_ref, dinv_ref, g0_ref):
    deg = jnp.sum(deg_ref[...], axis=0) + 1.0
    dinv = lax.rsqrt(deg)
    dinv_ref[...] = dinv
    g0_ref[...] = dinv * x_ref[...]


_scale0 = pl.pallas_call(
    _scale0_body,
    grid=(GRID,),
    in_specs=[
        pl.BlockSpec((NW, RB, 1), lambda i: (0, i, 0)),
        pl.BlockSpec((RB, D), lambda i: (i, 0)),
    ],
    out_specs=[
        pl.BlockSpec((RB, 1), lambda i: (i, 0)),
        pl.BlockSpec((RB, D), lambda i: (i, 0)),
    ],
    out_shape=[
        jax.ShapeDtypeStruct((N, 1), jnp.float32),
        jax.ShapeDtypeStruct((N, D), jnp.float32),
    ],
)


def _scale1_body(dinv_ref, pa_ref, pb_ref, g0_ref, g2_ref):
    dinv = dinv_ref[...]
    h = pa_ref[...] + pb_ref[...] + g0_ref[...]
    g2_ref[...] = h * (dinv * dinv)


_scale1 = pl.pallas_call(
    _scale1_body,
    grid=(GRID,),
    in_specs=[
        pl.BlockSpec((RB, 1), lambda i: (i, 0)),
        pl.BlockSpec((RB, D), lambda i: (i, 0)),
        pl.BlockSpec((RB, D), lambda i: (i, 0)),
        pl.BlockSpec((RB, D), lambda i: (i, 0)),
    ],
    out_specs=pl.BlockSpec((RB, D), lambda i: (i, 0)),
    out_shape=jax.ShapeDtypeStruct((N, D), jnp.float32),
)


def _final_body(dinv_ref, pa_ref, pb_ref, g2_ref, w_ref, b_ref, out_ref):
    h = dinv_ref[...] * (pa_ref[...] + pb_ref[...] + g2_ref[...])
    out_ref[...] = (
        jnp.dot(h, w_ref[...].T, preferred_element_type=jnp.float32)
        + b_ref[...]
    )


_final = pl.pallas_call(
    _final_body,
    grid=(GRID,),
    in_specs=[
        pl.BlockSpec((RB, 1), lambda i: (i, 0)),
        pl.BlockSpec((RB, D), lambda i: (i, 0)),
        pl.BlockSpec((RB, D), lambda i: (i, 0)),
        pl.BlockSpec((RB, D), lambda i: (i, 0)),
        pl.BlockSpec((C, D), lambda i: (0, 0)),
        pl.BlockSpec((1, C), lambda i: (0, 0)),
    ],
    out_specs=pl.BlockSpec((RB, C), lambda i: (i, 0)),
    out_shape=jax.ShapeDtypeStruct((N, C), jnp.float32),
)


def kernel(x, edge_index, W, b):
    ei = edge_index.astype(jnp.int32)
    row2 = ei[0].reshape(EROWS, CH)
    col2 = ei[1].reshape(EROWS, CH)
    zeros_2d = jnp.zeros((PT, D), jnp.float32)

    deg_parts = _deg_call(ei[1]).reshape(NW, NPAD, 1)
    dinv, g0 = _scale0(deg_parts, x)

    p = _prop_call(g0, row2, col2, zeros_2d)
    g2 = _scale1(dinv, p[0], p[1], g0)
    q = _prop_call(g2, row2, col2, zeros_2d)
    return _final(dinv, q[0], q[1], g2, W, b.reshape(1, C))
